# baseline (speedup 1.0000x reference)
import functools

import jax
import jax.numpy as jnp
from jax import lax
from jax.experimental import pallas as pl
from jax.experimental.pallas import tpu as pltpu
from jax.experimental.pallas import tpu_sc as plsc

N = 16 * 1 * 512 * 512
NC, NS, L = 2, 16, 16
NW = NC * NS
ROWS, COLS = 8192, 512
ROWS_W = ROWS // NW
CR = 32
NCHUNK = ROWS_W // CR
GRP = 4
RENORMS = ROWS_W * (COLS // (GRP * L))
EBIAS = RENORMS * 127
NB = 1024
BB = NB // NS
LMAX = 16.2
SCALE = NB / LMAX
LN2 = 0.6931471805599453
EPS = 1e-7
SQRT2 = 1.4142135
MANT = 0x007FFFFF
ONEB = 0x3F800000
_LOGC = (6.43456457838365e-08, 1.0000040910390389, -0.5000199361111282,
         0.33299593064817884, -0.24886355774399765, 0.20655376876344744,
         -0.18852653680148798, 0.11589704819807638)

_mesh = plsc.VectorSubcoreMesh(
    core_axis_name="c", subcore_axis_name="s", num_cores=NC, num_subcores=NS)
_cparams = pltpu.CompilerParams(needs_layout_passes=False)
_cparams_tc = pltpu.CompilerParams(needs_layout_passes=False,
                                   use_tc_tiling_on_sc=True,
                                   skip_device_barrier=True)


def _ln_mant(m, e_f):
    one = jnp.float32(1.0)
    big = m > jnp.float32(SQRT2)
    m = jnp.where(big, m * jnp.float32(0.5), m)
    e_f = e_f + jnp.where(big, one, jnp.float32(0.0))
    s = m - one
    pp = jnp.float32(_LOGC[7])
    for c in _LOGC[6::-1]:
        pp = pp * s + jnp.float32(c)
    return e_f * jnp.float32(LN2) + pp


def _loss16(p, t):
    one = jnp.float32(1.0)
    pc = jnp.minimum(jnp.maximum(p, jnp.float32(EPS)), one - jnp.float32(EPS))
    pos = t > jnp.float32(0.5)
    q = jnp.where(pos, pc, one - pc)
    bits = lax.bitcast_convert_type(q, jnp.int32)
    e = (bits >> 23) - 127
    m = lax.bitcast_convert_type((bits & MANT) | ONEB, jnp.float32)
    lnq = _ln_mant(m, e.astype(jnp.float32))
    return -lnq, pos


@functools.partial(
    pl.kernel,
    out_type=jax.ShapeDtypeStruct((NC * 128,), jnp.float32),
    mesh=_mesh,
    compiler_params=_cparams_tc,
    scratch_types=[
        pltpu.VMEM((CR, COLS), jnp.float32),
        pltpu.VMEM((CR, COLS), jnp.float32),
        pltpu.VMEM((CR, COLS), jnp.float32),
        pltpu.VMEM((CR, COLS), jnp.float32),
        pltpu.VMEM((2048,), jnp.float32),
        pltpu.VMEM((128,), jnp.float32),
        pltpu.VMEM_SHARED((NS * 128,), jnp.float32),
        pltpu.SemaphoreType.DMA,
        pltpu.SemaphoreType.DMA,
        pltpu.SemaphoreType.DMA,
        pltpu.SemaphoreType.DMA,
    ],
)
def _k1f(pred, tgt, outst,
         pbufA, tbufA, pbufB, tbufB, stage, statv, sh_stats,
         semPA, semTA, semPB, semTB):
    cid = lax.axis_index("c")
    sid = lax.axis_index("s")
    base = (cid * NS + sid) * ROWS_W
    z16 = jnp.zeros((L,), jnp.float32)
    one16 = jnp.full((L,), 1.0, jnp.float32)
    one = jnp.float32(1.0)

    def process(pref, tref, carry):
        def body(i, c):
            posc, mn, en, mp, ep = c
            r = i >> 3
            o = pl.multiple_of((i & 7) * (GRP * L), GRP * L)
            if True:
                qns, qps, ts = [], [], []
                for v in range(GRP):
                    p = pref[r, pl.ds(o + v * L, L)]
                    t = tref[r, pl.ds(o + v * L, L)]
                    pos = t > jnp.float32(0.5)
                    qps.append(jnp.maximum(jnp.where(pos, p, one16),
                                           jnp.float32(EPS)))
                    qns.append(jnp.maximum(jnp.where(pos, one16, one - p),
                                           jnp.float32(EPS)))
                    ts.append(t)
                posc = posc + ((ts[0] + ts[1]) + (ts[2] + ts[3]))
                pqn = (qns[0] * qns[1]) * (qns[2] * qns[3])
                pqp = (qps[0] * qps[1]) * (qps[2] * qps[3])
                mn = mn * pqn
                mp = mp * pqp
                bn = lax.bitcast_convert_type(mn, jnp.int32)
                en = en + (bn >> 23)
                mn = lax.bitcast_convert_type((bn & MANT) | ONEB, jnp.float32)
                bp = lax.bitcast_convert_type(mp, jnp.int32)
                ep = ep + (bp >> 23)
                mp = lax.bitcast_convert_type((bp & MANT) | ONEB, jnp.float32)
            return posc, mn, en, mp, ep
        return plsc.parallel_loop(0, CR * COLS // (GRP * L), unroll=2,
                                  carry=carry)(body)

    bufs = [(pbufA, tbufA, semPA, semTA), (pbufB, tbufB, semPB, semTB)]

    def issue(ci, slot):
        pb, tb, sp, st = bufs[slot]
        row = base + ci * CR
        pltpu.async_copy(pred.at[pl.ds(row, CR), :], pb, sp)
        pltpu.async_copy(tgt.at[pl.ds(row, CR), :], tb, st)

    def drain(slot):
        pb, tb, sp, st = bufs[slot]
        pltpu.make_async_copy(pred.at[pl.ds(0, CR), :], pb, sp).wait()
        pltpu.make_async_copy(tgt.at[pl.ds(0, CR), :], tb, st).wait()

    issue(0, 0)
    issue(1, 1)
    zi = jnp.zeros((L,), jnp.int32)
    carry = (z16, one16, zi, one16, zi)

    def pair_body(j, c):
        drain(0)
        c = process(pbufA, tbufA, c)
        issue(jnp.minimum(2 * j + 2, NCHUNK - 1), 0)
        drain(1)
        c = process(pbufB, tbufB, c)
        issue(jnp.minimum(2 * j + 3, NCHUNK - 1), 1)
        return c

    carry = lax.fori_loop(0, NCHUNK // 2, pair_body, carry)
    drain(0)
    drain(1)
    posc, mn, en, mp, ep = carry

    negs = -_ln_mant(mn, (en - EBIAS).astype(jnp.float32))
    poss = -_ln_mant(mp, (ep - EBIAS).astype(jnp.float32))
    alls = poss + negs

    statv[pl.ds(0, L)] = posc
    statv[pl.ds(16, L)] = poss
    statv[pl.ds(32, L)] = alls
    for v in range(3, 8):
        statv[pl.ds(v * 16, L)] = z16
    pltpu.sync_copy(statv, sh_stats.at[pl.ds(sid * 128, 128)])
    plsc.subcore_barrier()

    @pl.when(sid == 0)
    def _():
        pltpu.sync_copy(sh_stats, stage.at[pl.ds(0, NS * 128)])
        for v in range(3):
            acc = z16
            for r in range(NS):
                acc = acc + stage[pl.ds(r * 128 + v * 16, L)]
            statv[pl.ds(v * 16, L)] = acc
        pltpu.sync_copy(statv, outst.at[pl.ds(cid * 128, 128)])


@functools.partial(
    pl.kernel,
    out_type=(
        jax.ShapeDtypeStruct((NC * NB,), jnp.float32),
        jax.ShapeDtypeStruct((NC * NB,), jnp.float32),
        jax.ShapeDtypeStruct((NC * 128,), jnp.float32),
    ),
    mesh=_mesh,
    compiler_params=_cparams_tc,
    scratch_types=[
        pltpu.VMEM((CR, COLS), jnp.float32),
        pltpu.VMEM((CR, COLS), jnp.float32),
        pltpu.VMEM((CR, COLS), jnp.float32),
        pltpu.VMEM((CR, COLS), jnp.float32),
        pltpu.VMEM((NB,), jnp.float32),
        pltpu.VMEM((NB,), jnp.float32),
        pltpu.VMEM((BB,), jnp.float32),
        pltpu.VMEM((BB,), jnp.float32),
        pltpu.VMEM((2048,), jnp.float32),
        pltpu.VMEM((128,), jnp.float32),
        pltpu.VMEM_SHARED((NS * NB,), jnp.float32),
        pltpu.VMEM_SHARED((NS * NB,), jnp.float32),
        pltpu.VMEM_SHARED((NB,), jnp.float32),
        pltpu.VMEM_SHARED((NB,), jnp.float32),
        pltpu.VMEM_SHARED((NS * 128,), jnp.float32),
        pltpu.SemaphoreType.DMA,
        pltpu.SemaphoreType.DMA,
        pltpu.SemaphoreType.DMA,
        pltpu.SemaphoreType.DMA,
    ],
)
def _k1(pred, tgt, outc, outs, outst,
        pbufA, tbufA, pbufB, tbufB, histc_v, hists_v, accc, accs, stage, statv,
        sh_histc, sh_hists, sh_redc, sh_reds, sh_stats,
        semPA, semTA, semPB, semTB):
    cid = lax.axis_index("c")
    sid = lax.axis_index("s")
    base = (cid * NS + sid) * ROWS_W
    z16 = jnp.zeros((L,), jnp.float32)
    ones16 = jnp.full((L,), 1.0, jnp.float32)

    def _zi(i, _):
        o = pl.multiple_of(i * L, L)
        histc_v[pl.ds(o, L)] = z16
        hists_v[pl.ds(o, L)] = z16
        return 0
    lax.fori_loop(0, NB // L, _zi, 0)

    def process(pref, tref, carry):
        def body(j, c):
            posc, poss, alls = c
            r = j >> 5
            o = pl.multiple_of((j & 31) * L, L)
            if True:
                p = pref[r, pl.ds(o, L)]
                t = tref[r, pl.ds(o, L)]
                loss, pos = _loss16(p, t)
                alls = alls + loss
                posc = posc + t
                poss = poss + jnp.where(pos, loss, jnp.float32(0.0))
                binv = jnp.minimum(
                    (loss * jnp.float32(SCALE)).astype(jnp.int32), NB - 1)
                negm = jnp.logical_not(pos)
                plsc.addupdate_scatter(histc_v, [binv], ones16, mask=negm)
                plsc.addupdate_scatter(hists_v, [binv], loss, mask=negm)
            return posc, poss, alls
        return plsc.parallel_loop(0, CR * COLS // L, unroll=4,
                                  carry=carry)(body)

    bufs = [(pbufA, tbufA, semPA, semTA), (pbufB, tbufB, semPB, semTB)]

    def issue(ci, slot):
        pb, tb, sp, st = bufs[slot]
        row = base + ci * CR
        pltpu.async_copy(pred.at[pl.ds(row, CR), :], pb, sp)
        pltpu.async_copy(tgt.at[pl.ds(row, CR), :], tb, st)

    def drain(slot):
        pb, tb, sp, st = bufs[slot]
        pltpu.make_async_copy(pred.at[pl.ds(0, CR), :], pb, sp).wait()
        pltpu.make_async_copy(tgt.at[pl.ds(0, CR), :], tb, st).wait()

    issue(0, 0)
    issue(1, 1)
    carry = (z16, z16, z16)

    def pair_body(j, c):
        drain(0)
        c = process(pbufA, tbufA, c)
        issue(jnp.minimum(2 * j + 2, NCHUNK - 1), 0)
        drain(1)
        c = process(pbufB, tbufB, c)
        issue(jnp.minimum(2 * j + 3, NCHUNK - 1), 1)
        return c

    carry = lax.fori_loop(0, NCHUNK // 2, pair_body, carry)
    drain(0)
    drain(1)
    posc, poss, alls = carry

    statv[pl.ds(0, L)] = posc
    statv[pl.ds(16, L)] = poss
    statv[pl.ds(32, L)] = alls
    for v in range(3, 8):
        statv[pl.ds(v * 16, L)] = z16
    pltpu.sync_copy(statv, sh_stats.at[pl.ds(sid * 128, 128)])
    pltpu.sync_copy(histc_v, sh_histc.at[pl.ds(sid * NB, NB)])
    pltpu.sync_copy(hists_v, sh_hists.at[pl.ds(sid * NB, NB)])
    plsc.subcore_barrier()

    for r in range(NS):
        pltpu.sync_copy(sh_histc.at[pl.ds(r * NB + sid * BB, BB)],
                        histc_v.at[pl.ds(r * BB, BB)])
        pltpu.sync_copy(sh_hists.at[pl.ds(r * NB + sid * BB, BB)],
                        hists_v.at[pl.ds(r * BB, BB)])

    def _red(v, _):
        o = pl.multiple_of(v * L, L)
        cacc = z16
        sacc = z16
        for r in range(NS):
            cacc = cacc + histc_v[pl.ds(r * BB + o, L)]
            sacc = sacc + hists_v[pl.ds(r * BB + o, L)]
        accc[pl.ds(o, L)] = cacc
        accs[pl.ds(o, L)] = sacc
        return 0
    lax.fori_loop(0, BB // L, _red, 0)
    pltpu.sync_copy(accc, sh_redc.at[pl.ds(sid * BB, BB)])
    pltpu.sync_copy(accs, sh_reds.at[pl.ds(sid * BB, BB)])
    plsc.subcore_barrier()

    @pl.when(sid == 0)
    def _():
        pltpu.sync_copy(sh_redc, outc.at[pl.ds(cid * NB, NB)])
        pltpu.sync_copy(sh_reds, outs.at[pl.ds(cid * NB, NB)])
        pltpu.sync_copy(sh_stats, stage.at[pl.ds(0, NS * 128)])
        for v in range(3):
            acc = z16
            for r in range(NS):
                acc = acc + stage[pl.ds(r * 128 + v * 16, L)]
            statv[pl.ds(v * 16, L)] = acc
        pltpu.sync_copy(statv, outst.at[pl.ds(cid * 128, 128)])


@functools.partial(
    pl.kernel,
    out_type=jax.ShapeDtypeStruct((L,), jnp.float32),
    mesh=_mesh,
    compiler_params=_cparams,
    scratch_types=[
        pltpu.VMEM((NB,), jnp.float32),
        pltpu.VMEM((NB,), jnp.float32),
        pltpu.VMEM((NB,), jnp.float32),
        pltpu.VMEM((NB,), jnp.float32),
        pltpu.VMEM((256,), jnp.float32),
        pltpu.VMEM((L,), jnp.float32),
    ],
)
def _k2(histc, hists, stats, out, c0, c1, s0, s1, st_v, outbuf):
    cid = lax.axis_index("c")
    sid = lax.axis_index("s")

    @pl.when(jnp.logical_and(cid == 0, sid == 0))
    def _():
        pltpu.sync_copy(histc.at[pl.ds(0, NB)], c0)
        pltpu.sync_copy(histc.at[pl.ds(NB, NB)], c1)
        pltpu.sync_copy(hists.at[pl.ds(0, NB)], s0)
        pltpu.sync_copy(hists.at[pl.ds(NB, NB)], s1)
        pltpu.sync_copy(stats, st_v)

        pos_cnt = jnp.sum(st_v[pl.ds(0, L)] + st_v[pl.ds(128, L)])
        pos_sum = jnp.sum(st_v[pl.ds(16, L)] + st_v[pl.ds(144, L)])
        all_sum = jnp.sum(st_v[pl.ds(32, L)] + st_v[pl.ds(160, L)])
        neg_cnt = jnp.float32(N) - pos_cnt
        neg_sum = all_sum - pos_sum
        k = jnp.minimum(neg_cnt, jnp.float32(3.0) * pos_cnt)

        def sel_body(jj, carry):
            above, sel = carry
            o = (NB // L - 1 - jj) * L
            cv = c0[pl.ds(o, L)] + c1[pl.ds(o, L)]
            sv = s0[pl.ds(o, L)] + s1[pl.ds(o, L)]
            pc = plsc.cumsum(cv)
            tot = jnp.sum(cv)
            above_i = above + (tot - pc)
            take = jnp.minimum(jnp.maximum(k - above_i, jnp.float32(0.0)), cv)
            avg = sv / jnp.maximum(cv, jnp.float32(1.0))
            sel = sel + jnp.sum(take * avg)
            return above + tot, sel

        _, sel = lax.fori_loop(0, NB // L, sel_body,
                               (jnp.float32(0.0), jnp.float32(0.0)))
        neg_loss = jnp.where(k >= neg_cnt, neg_sum, sel)
        total = pos_cnt + k
        ones_v = jnp.full((L,), 1.0, jnp.float32)
        num_v = ones_v * (pos_sum + neg_loss)
        den_v = ones_v * jnp.maximum(total, jnp.float32(1.0))
        res_v = num_v / den_v
        outbuf[...] = jnp.where(ones_v * total > jnp.float32(0.0), res_v,
                                jnp.zeros((L,), jnp.float32))
        pltpu.sync_copy(outbuf, out)


def kernel(pred, target):
    p = pred.reshape(ROWS, COLS)
    t = target.reshape(ROWS, COLS)
    stats = _k1f(p, t).reshape(NC, 128)
    pos_cnt = jnp.sum(stats[:, 0:16])
    pos_sum = jnp.sum(stats[:, 16:32])
    all_sum = jnp.sum(stats[:, 32:48])
    neg_cnt = jnp.float32(N) - pos_cnt
    neg_sum = all_sum - pos_sum
    k = jnp.minimum(neg_cnt, jnp.floor(jnp.float32(3.0) * pos_cnt))
    res = (pos_sum + neg_sum) / jnp.float32(N)
    need_sel = k < neg_cnt

    def _slow():
        hc, hs, st = _k1(p, t)
        return _k2(hc, hs, st)[0]

    return lax.cond(need_sel, _slow, lambda: res)


# speedup vs baseline: 1.1372x; 1.1372x over previous
import functools

import jax
import jax.numpy as jnp
from jax import lax
from jax.experimental import pallas as pl
from jax.experimental.pallas import tpu as pltpu
from jax.experimental.pallas import tpu_sc as plsc

N = 16 * 1 * 512 * 512
NC, NS, L = 2, 16, 16
NW = NC * NS
ROWS, COLS = 8192, 512
SC_ROWS = 4096
TC_ROWS = ROWS - SC_ROWS
ROWS_W = ROWS // NW
FROWS_W = SC_ROWS // NW
CR = 32
NCHUNK = ROWS_W // CR
FNCHUNK = FROWS_W // CR
GRP = 4
RENORMS = FROWS_W * (COLS // (GRP * L))
EBIAS = RENORMS * 127
NB = 1024
BB = NB // NS
LMAX = 16.2
SCALE = NB / LMAX
LN2 = 0.6931471805599453
EPS = 1e-7
SQRT2 = 1.4142135
MANT = 0x007FFFFF
ONEB = 0x3F800000
_LOGC = (6.43456457838365e-08, 1.0000040910390389, -0.5000199361111282,
         0.33299593064817884, -0.24886355774399765, 0.20655376876344744,
         -0.18852653680148798, 0.11589704819807638)

_mesh = plsc.VectorSubcoreMesh(
    core_axis_name="c", subcore_axis_name="s", num_cores=NC, num_subcores=NS)
_cparams = pltpu.CompilerParams(needs_layout_passes=False)
_cparams_tc = pltpu.CompilerParams(needs_layout_passes=False,
                                   use_tc_tiling_on_sc=True)


def _ln_mant(m, e_f):
    one = jnp.float32(1.0)
    big = m > jnp.float32(SQRT2)
    m = jnp.where(big, m * jnp.float32(0.5), m)
    e_f = e_f + jnp.where(big, one, jnp.float32(0.0))
    s = m - one
    pp = jnp.float32(_LOGC[7])
    for c in _LOGC[6::-1]:
        pp = pp * s + jnp.float32(c)
    return e_f * jnp.float32(LN2) + pp


def _loss16(p, t):
    one = jnp.float32(1.0)
    pc = jnp.minimum(jnp.maximum(p, jnp.float32(EPS)), one - jnp.float32(EPS))
    pos = t > jnp.float32(0.5)
    q = jnp.where(pos, pc, one - pc)
    bits = lax.bitcast_convert_type(q, jnp.int32)
    e = (bits >> 23) - 127
    m = lax.bitcast_convert_type((bits & MANT) | ONEB, jnp.float32)
    lnq = _ln_mant(m, e.astype(jnp.float32))
    return -lnq, pos


@functools.partial(
    pl.kernel,
    out_type=jax.ShapeDtypeStruct((NC * 128,), jnp.float32),
    mesh=_mesh,
    compiler_params=_cparams_tc,
    scratch_types=[
        pltpu.VMEM((CR, COLS), jnp.float32),
        pltpu.VMEM((CR, COLS), jnp.float32),
        pltpu.VMEM((CR, COLS), jnp.float32),
        pltpu.VMEM((CR, COLS), jnp.float32),
        pltpu.VMEM((2048,), jnp.float32),
        pltpu.VMEM((128,), jnp.float32),
        pltpu.VMEM_SHARED((NS * 128,), jnp.float32),
        pltpu.SemaphoreType.DMA,
        pltpu.SemaphoreType.DMA,
        pltpu.SemaphoreType.DMA,
        pltpu.SemaphoreType.DMA,
    ],
)
def _k1f(pred, tgt, outst,
         pbufA, tbufA, pbufB, tbufB, stage, statv, sh_stats,
         semPA, semTA, semPB, semTB):
    cid = lax.axis_index("c")
    sid = lax.axis_index("s")
    base = (cid * NS + sid) * FROWS_W
    z16 = jnp.zeros((L,), jnp.float32)
    one16 = jnp.full((L,), 1.0, jnp.float32)
    one = jnp.float32(1.0)

    def process(pref, tref, carry):
        def body(i, c):
            posc, mn, en, mp, ep = c
            r = i >> 3
            o = pl.multiple_of((i & 7) * (GRP * L), GRP * L)
            if True:
                qns, qps, ts = [], [], []
                for v in range(GRP):
                    p = pref[r, pl.ds(o + v * L, L)]
                    t = tref[r, pl.ds(o + v * L, L)]
                    pos = t > jnp.float32(0.5)
                    qps.append(jnp.maximum(jnp.where(pos, p, one16),
                                           jnp.float32(EPS)))
                    qns.append(jnp.maximum(jnp.where(pos, one16, one - p),
                                           jnp.float32(EPS)))
                    ts.append(t)
                posc = posc + ((ts[0] + ts[1]) + (ts[2] + ts[3]))
                pqn = (qns[0] * qns[1]) * (qns[2] * qns[3])
                pqp = (qps[0] * qps[1]) * (qps[2] * qps[3])
                mn = mn * pqn
                mp = mp * pqp
                bn = lax.bitcast_convert_type(mn, jnp.int32)
                en = en + (bn >> 23)
                mn = lax.bitcast_convert_type((bn & MANT) | ONEB, jnp.float32)
                bp = lax.bitcast_convert_type(mp, jnp.int32)
                ep = ep + (bp >> 23)
                mp = lax.bitcast_convert_type((bp & MANT) | ONEB, jnp.float32)
            return posc, mn, en, mp, ep
        return plsc.parallel_loop(0, CR * COLS // (GRP * L), unroll=2,
                                  carry=carry)(body)

    bufs = [(pbufA, tbufA, semPA, semTA), (pbufB, tbufB, semPB, semTB)]

    def issue(ci, slot):
        pb, tb, sp, st = bufs[slot]
        row = base + ci * CR
        pltpu.async_copy(pred.at[pl.ds(row, CR), :], pb, sp)
        pltpu.async_copy(tgt.at[pl.ds(row, CR), :], tb, st)

    def drain(slot):
        pb, tb, sp, st = bufs[slot]
        pltpu.make_async_copy(pred.at[pl.ds(0, CR), :], pb, sp).wait()
        pltpu.make_async_copy(tgt.at[pl.ds(0, CR), :], tb, st).wait()

    issue(0, 0)
    issue(1, 1)
    zi = jnp.zeros((L,), jnp.int32)
    carry = (z16, one16, zi, one16, zi)

    def pair_body(j, c):
        drain(0)
        c = process(pbufA, tbufA, c)
        issue(jnp.minimum(2 * j + 2, FNCHUNK - 1), 0)
        drain(1)
        c = process(pbufB, tbufB, c)
        issue(jnp.minimum(2 * j + 3, FNCHUNK - 1), 1)
        return c

    carry = lax.fori_loop(0, FNCHUNK // 2, pair_body, carry)
    drain(0)
    drain(1)
    posc, mn, en, mp, ep = carry

    negs = -_ln_mant(mn, (en - EBIAS).astype(jnp.float32))
    poss = -_ln_mant(mp, (ep - EBIAS).astype(jnp.float32))
    alls = poss + negs

    statv[pl.ds(0, L)] = posc
    statv[pl.ds(16, L)] = poss
    statv[pl.ds(32, L)] = alls
    for v in range(3, 8):
        statv[pl.ds(v * 16, L)] = z16
    pltpu.sync_copy(statv, sh_stats.at[pl.ds(sid * 128, 128)])
    plsc.subcore_barrier()

    @pl.when(sid == 0)
    def _():
        pltpu.sync_copy(sh_stats, stage.at[pl.ds(0, NS * 128)])
        for v in range(3):
            acc = z16
            for r in range(NS):
                acc = acc + stage[pl.ds(r * 128 + v * 16, L)]
            statv[pl.ds(v * 16, L)] = acc
        pltpu.sync_copy(statv, outst.at[pl.ds(cid * 128, 128)])


@functools.partial(
    pl.kernel,
    out_type=(
        jax.ShapeDtypeStruct((NC * NB,), jnp.float32),
        jax.ShapeDtypeStruct((NC * NB,), jnp.float32),
        jax.ShapeDtypeStruct((NC * 128,), jnp.float32),
    ),
    mesh=_mesh,
    compiler_params=_cparams_tc,
    scratch_types=[
        pltpu.VMEM((CR, COLS), jnp.float32),
        pltpu.VMEM((CR, COLS), jnp.float32),
        pltpu.VMEM((CR, COLS), jnp.float32),
        pltpu.VMEM((CR, COLS), jnp.float32),
        pltpu.VMEM((NB,), jnp.float32),
        pltpu.VMEM((NB,), jnp.float32),
        pltpu.VMEM((BB,), jnp.float32),
        pltpu.VMEM((BB,), jnp.float32),
        pltpu.VMEM((2048,), jnp.float32),
        pltpu.VMEM((128,), jnp.float32),
        pltpu.VMEM_SHARED((NS * NB,), jnp.float32),
        pltpu.VMEM_SHARED((NS * NB,), jnp.float32),
        pltpu.VMEM_SHARED((NB,), jnp.float32),
        pltpu.VMEM_SHARED((NB,), jnp.float32),
        pltpu.VMEM_SHARED((NS * 128,), jnp.float32),
        pltpu.SemaphoreType.DMA,
        pltpu.SemaphoreType.DMA,
        pltpu.SemaphoreType.DMA,
        pltpu.SemaphoreType.DMA,
    ],
)
def _k1(pred, tgt, outc, outs, outst,
        pbufA, tbufA, pbufB, tbufB, histc_v, hists_v, accc, accs, stage, statv,
        sh_histc, sh_hists, sh_redc, sh_reds, sh_stats,
        semPA, semTA, semPB, semTB):
    cid = lax.axis_index("c")
    sid = lax.axis_index("s")
    base = (cid * NS + sid) * ROWS_W
    z16 = jnp.zeros((L,), jnp.float32)
    ones16 = jnp.full((L,), 1.0, jnp.float32)

    def _zi(i, _):
        o = pl.multiple_of(i * L, L)
        histc_v[pl.ds(o, L)] = z16
        hists_v[pl.ds(o, L)] = z16
        return 0
    lax.fori_loop(0, NB // L, _zi, 0)

    def process(pref, tref, carry):
        def body(j, c):
            posc, poss, alls = c
            r = j >> 5
            o = pl.multiple_of((j & 31) * L, L)
            if True:
                p = pref[r, pl.ds(o, L)]
                t = tref[r, pl.ds(o, L)]
                loss, pos = _loss16(p, t)
                alls = alls + loss
                posc = posc + t
                poss = poss + jnp.where(pos, loss, jnp.float32(0.0))
                binv = jnp.minimum(
                    (loss * jnp.float32(SCALE)).astype(jnp.int32), NB - 1)
                negm = jnp.logical_not(pos)
                plsc.addupdate_scatter(histc_v, [binv], ones16, mask=negm)
                plsc.addupdate_scatter(hists_v, [binv], loss, mask=negm)
            return posc, poss, alls
        return plsc.parallel_loop(0, CR * COLS // L, unroll=4,
                                  carry=carry)(body)

    bufs = [(pbufA, tbufA, semPA, semTA), (pbufB, tbufB, semPB, semTB)]

    def issue(ci, slot):
        pb, tb, sp, st = bufs[slot]
        row = base + ci * CR
        pltpu.async_copy(pred.at[pl.ds(row, CR), :], pb, sp)
        pltpu.async_copy(tgt.at[pl.ds(row, CR), :], tb, st)

    def drain(slot):
        pb, tb, sp, st = bufs[slot]
        pltpu.make_async_copy(pred.at[pl.ds(0, CR), :], pb, sp).wait()
        pltpu.make_async_copy(tgt.at[pl.ds(0, CR), :], tb, st).wait()

    issue(0, 0)
    issue(1, 1)
    carry = (z16, z16, z16)

    def pair_body(j, c):
        drain(0)
        c = process(pbufA, tbufA, c)
        issue(jnp.minimum(2 * j + 2, NCHUNK - 1), 0)
        drain(1)
        c = process(pbufB, tbufB, c)
        issue(jnp.minimum(2 * j + 3, NCHUNK - 1), 1)
        return c

    carry = lax.fori_loop(0, NCHUNK // 2, pair_body, carry)
    drain(0)
    drain(1)
    posc, poss, alls = carry

    statv[pl.ds(0, L)] = posc
    statv[pl.ds(16, L)] = poss
    statv[pl.ds(32, L)] = alls
    for v in range(3, 8):
        statv[pl.ds(v * 16, L)] = z16
    pltpu.sync_copy(statv, sh_stats.at[pl.ds(sid * 128, 128)])
    pltpu.sync_copy(histc_v, sh_histc.at[pl.ds(sid * NB, NB)])
    pltpu.sync_copy(hists_v, sh_hists.at[pl.ds(sid * NB, NB)])
    plsc.subcore_barrier()

    for r in range(NS):
        pltpu.sync_copy(sh_histc.at[pl.ds(r * NB + sid * BB, BB)],
                        histc_v.at[pl.ds(r * BB, BB)])
        pltpu.sync_copy(sh_hists.at[pl.ds(r * NB + sid * BB, BB)],
                        hists_v.at[pl.ds(r * BB, BB)])

    def _red(v, _):
        o = pl.multiple_of(v * L, L)
        cacc = z16
        sacc = z16
        for r in range(NS):
            cacc = cacc + histc_v[pl.ds(r * BB + o, L)]
            sacc = sacc + hists_v[pl.ds(r * BB + o, L)]
        accc[pl.ds(o, L)] = cacc
        accs[pl.ds(o, L)] = sacc
        return 0
    lax.fori_loop(0, BB // L, _red, 0)
    pltpu.sync_copy(accc, sh_redc.at[pl.ds(sid * BB, BB)])
    pltpu.sync_copy(accs, sh_reds.at[pl.ds(sid * BB, BB)])
    plsc.subcore_barrier()

    @pl.when(sid == 0)
    def _():
        pltpu.sync_copy(sh_redc, outc.at[pl.ds(cid * NB, NB)])
        pltpu.sync_copy(sh_reds, outs.at[pl.ds(cid * NB, NB)])
        pltpu.sync_copy(sh_stats, stage.at[pl.ds(0, NS * 128)])
        for v in range(3):
            acc = z16
            for r in range(NS):
                acc = acc + stage[pl.ds(r * 128 + v * 16, L)]
            statv[pl.ds(v * 16, L)] = acc
        pltpu.sync_copy(statv, outst.at[pl.ds(cid * 128, 128)])


@functools.partial(
    pl.kernel,
    out_type=jax.ShapeDtypeStruct((L,), jnp.float32),
    mesh=_mesh,
    compiler_params=_cparams,
    scratch_types=[
        pltpu.VMEM((NB,), jnp.float32),
        pltpu.VMEM((NB,), jnp.float32),
        pltpu.VMEM((NB,), jnp.float32),
        pltpu.VMEM((NB,), jnp.float32),
        pltpu.VMEM((256,), jnp.float32),
        pltpu.VMEM((L,), jnp.float32),
    ],
)
def _k2(histc, hists, stats, out, c0, c1, s0, s1, st_v, outbuf):
    cid = lax.axis_index("c")
    sid = lax.axis_index("s")

    @pl.when(jnp.logical_and(cid == 0, sid == 0))
    def _():
        pltpu.sync_copy(histc.at[pl.ds(0, NB)], c0)
        pltpu.sync_copy(histc.at[pl.ds(NB, NB)], c1)
        pltpu.sync_copy(hists.at[pl.ds(0, NB)], s0)
        pltpu.sync_copy(hists.at[pl.ds(NB, NB)], s1)
        pltpu.sync_copy(stats, st_v)

        pos_cnt = jnp.sum(st_v[pl.ds(0, L)] + st_v[pl.ds(128, L)])
        pos_sum = jnp.sum(st_v[pl.ds(16, L)] + st_v[pl.ds(144, L)])
        all_sum = jnp.sum(st_v[pl.ds(32, L)] + st_v[pl.ds(160, L)])
        neg_cnt = jnp.float32(N) - pos_cnt
        neg_sum = all_sum - pos_sum
        k = jnp.minimum(neg_cnt, jnp.float32(3.0) * pos_cnt)

        def sel_body(jj, carry):
            above, sel = carry
            o = (NB // L - 1 - jj) * L
            cv = c0[pl.ds(o, L)] + c1[pl.ds(o, L)]
            sv = s0[pl.ds(o, L)] + s1[pl.ds(o, L)]
            pc = plsc.cumsum(cv)
            tot = jnp.sum(cv)
            above_i = above + (tot - pc)
            take = jnp.minimum(jnp.maximum(k - above_i, jnp.float32(0.0)), cv)
            avg = sv / jnp.maximum(cv, jnp.float32(1.0))
            sel = sel + jnp.sum(take * avg)
            return above + tot, sel

        _, sel = lax.fori_loop(0, NB // L, sel_body,
                               (jnp.float32(0.0), jnp.float32(0.0)))
        neg_loss = jnp.where(k >= neg_cnt, neg_sum, sel)
        total = pos_cnt + k
        ones_v = jnp.full((L,), 1.0, jnp.float32)
        num_v = ones_v * (pos_sum + neg_loss)
        den_v = ones_v * jnp.maximum(total, jnp.float32(1.0))
        res_v = num_v / den_v
        outbuf[...] = jnp.where(ones_v * total > jnp.float32(0.0), res_v,
                                jnp.zeros((L,), jnp.float32))
        pltpu.sync_copy(outbuf, out)


_TCG = 8
_TBR = TC_ROWS // _TCG


def _tc_body(x_ref, t_ref, oc_ref, op_ref, oa_ref):
    i = pl.program_id(0)

    @pl.when(i == 0)
    def _():
        oc_ref[...] = jnp.zeros_like(oc_ref)
        op_ref[...] = jnp.zeros_like(op_ref)
        oa_ref[...] = jnp.zeros_like(oa_ref)

    p = x_ref[...]
    t = t_ref[...]
    pc = jnp.clip(p, jnp.float32(EPS), jnp.float32(1.0) - jnp.float32(EPS))
    pos = t > jnp.float32(0.5)
    q = jnp.where(pos, pc, jnp.float32(1.0) - pc)
    loss = -jnp.log(q)
    oc_ref[...] += jnp.sum(t).reshape(1, 1)
    op_ref[...] += jnp.sum(jnp.where(pos, loss, jnp.float32(0.0))).reshape(1, 1)
    oa_ref[...] += jnp.sum(loss).reshape(1, 1)


_s11 = jax.ShapeDtypeStruct((1, 1), jnp.float32)
_tcsum = pl.pallas_call(
    _tc_body,
    grid=(_TCG,),
    in_specs=[
        pl.BlockSpec((_TBR, COLS), lambda i: (SC_ROWS // _TBR + i, 0)),
        pl.BlockSpec((_TBR, COLS), lambda i: (SC_ROWS // _TBR + i, 0)),
    ],
    out_specs=(
        pl.BlockSpec((1, 1), lambda i: (0, 0)),
        pl.BlockSpec((1, 1), lambda i: (0, 0)),
        pl.BlockSpec((1, 1), lambda i: (0, 0)),
    ),
    out_shape=(_s11, _s11, _s11),
)


def kernel(pred, target):
    p = pred.reshape(ROWS, COLS)
    t = target.reshape(ROWS, COLS)
    stats = _k1f(p, t).reshape(NC, 128)
    tc_cnt, tc_pos, tc_all = _tcsum(p, t)
    pos_cnt = jnp.sum(stats[:, 0:16]) + tc_cnt[0, 0]
    pos_sum = jnp.sum(stats[:, 16:32]) + tc_pos[0, 0]
    all_sum = jnp.sum(stats[:, 32:48]) + tc_all[0, 0]
    neg_cnt = jnp.float32(N) - pos_cnt
    neg_sum = all_sum - pos_sum
    k = jnp.minimum(neg_cnt, jnp.floor(jnp.float32(3.0) * pos_cnt))
    res = (pos_sum + neg_sum) / jnp.float32(N)
    need_sel = k < neg_cnt

    def _slow():
        hc, hs, st = _k1(p, t)
        return _k2(hc, hs, st)[0]

    return lax.cond(need_sel, _slow, lambda: res)


# speedup vs baseline: 1.2162x; 1.0694x over previous
import functools

import jax
import jax.numpy as jnp
from jax import lax
from jax.experimental import pallas as pl
from jax.experimental.pallas import tpu as pltpu
from jax.experimental.pallas import tpu_sc as plsc

N = 16 * 1 * 512 * 512
NC, NS, L = 2, 16, 16
NW = NC * NS
ROWS, COLS = 8192, 512
SC_ROWS = 2048
TC_ROWS = ROWS - SC_ROWS
ROWS_W = ROWS // NW
FROWS_W = SC_ROWS // NW
CR = 32
NCHUNK = ROWS_W // CR
FNCHUNK = FROWS_W // CR
GRP = 4
RENORMS = FROWS_W * (COLS // (GRP * L))
EBIAS = RENORMS * 127
NB = 1024
BB = NB // NS
LMAX = 16.2
SCALE = NB / LMAX
LN2 = 0.6931471805599453
EPS = 1e-7
SQRT2 = 1.4142135
MANT = 0x007FFFFF
ONEB = 0x3F800000
_LOGC = (6.43456457838365e-08, 1.0000040910390389, -0.5000199361111282,
         0.33299593064817884, -0.24886355774399765, 0.20655376876344744,
         -0.18852653680148798, 0.11589704819807638)

_mesh = plsc.VectorSubcoreMesh(
    core_axis_name="c", subcore_axis_name="s", num_cores=NC, num_subcores=NS)
_cparams = pltpu.CompilerParams(needs_layout_passes=False)
_cparams_tc = pltpu.CompilerParams(needs_layout_passes=False,
                                   use_tc_tiling_on_sc=True)


def _ln_mant(m, e_f):
    one = jnp.float32(1.0)
    big = m > jnp.float32(SQRT2)
    m = jnp.where(big, m * jnp.float32(0.5), m)
    e_f = e_f + jnp.where(big, one, jnp.float32(0.0))
    s = m - one
    pp = jnp.float32(_LOGC[7])
    for c in _LOGC[6::-1]:
        pp = pp * s + jnp.float32(c)
    return e_f * jnp.float32(LN2) + pp


def _loss16(p, t):
    one = jnp.float32(1.0)
    pc = jnp.minimum(jnp.maximum(p, jnp.float32(EPS)), one - jnp.float32(EPS))
    pos = t > jnp.float32(0.5)
    q = jnp.where(pos, pc, one - pc)
    bits = lax.bitcast_convert_type(q, jnp.int32)
    e = (bits >> 23) - 127
    m = lax.bitcast_convert_type((bits & MANT) | ONEB, jnp.float32)
    lnq = _ln_mant(m, e.astype(jnp.float32))
    return -lnq, pos


@functools.partial(
    pl.kernel,
    out_type=jax.ShapeDtypeStruct((NC * 128,), jnp.float32),
    mesh=_mesh,
    compiler_params=_cparams_tc,
    scratch_types=[
        pltpu.VMEM((CR, COLS), jnp.float32),
        pltpu.VMEM((CR, COLS), jnp.float32),
        pltpu.VMEM((CR, COLS), jnp.float32),
        pltpu.VMEM((CR, COLS), jnp.float32),
        pltpu.VMEM((2048,), jnp.float32),
        pltpu.VMEM((128,), jnp.float32),
        pltpu.VMEM_SHARED((NS * 128,), jnp.float32),
        pltpu.SemaphoreType.DMA,
        pltpu.SemaphoreType.DMA,
        pltpu.SemaphoreType.DMA,
        pltpu.SemaphoreType.DMA,
    ],
)
def _k1f(pred, tgt, outst,
         pbufA, tbufA, pbufB, tbufB, stage, statv, sh_stats,
         semPA, semTA, semPB, semTB):
    cid = lax.axis_index("c")
    sid = lax.axis_index("s")
    base = (cid * NS + sid) * FROWS_W
    z16 = jnp.zeros((L,), jnp.float32)
    one16 = jnp.full((L,), 1.0, jnp.float32)
    one = jnp.float32(1.0)

    def process(pref, tref, carry):
        def body(i, c):
            posc, mn, en, mp, ep = c
            r = i >> 3
            o = pl.multiple_of((i & 7) * (GRP * L), GRP * L)
            if True:
                qns, qps, ts = [], [], []
                for v in range(GRP):
                    p = pref[r, pl.ds(o + v * L, L)]
                    t = tref[r, pl.ds(o + v * L, L)]
                    pos = t > jnp.float32(0.5)
                    qps.append(jnp.maximum(jnp.where(pos, p, one16),
                                           jnp.float32(EPS)))
                    qns.append(jnp.maximum(jnp.where(pos, one16, one - p),
                                           jnp.float32(EPS)))
                    ts.append(t)
                posc = posc + ((ts[0] + ts[1]) + (ts[2] + ts[3]))
                pqn = (qns[0] * qns[1]) * (qns[2] * qns[3])
                pqp = (qps[0] * qps[1]) * (qps[2] * qps[3])
                mn = mn * pqn
                mp = mp * pqp
                bn = lax.bitcast_convert_type(mn, jnp.int32)
                en = en + (bn >> 23)
                mn = lax.bitcast_convert_type((bn & MANT) | ONEB, jnp.float32)
                bp = lax.bitcast_convert_type(mp, jnp.int32)
                ep = ep + (bp >> 23)
                mp = lax.bitcast_convert_type((bp & MANT) | ONEB, jnp.float32)
            return posc, mn, en, mp, ep
        return plsc.parallel_loop(0, CR * COLS // (GRP * L), unroll=2,
                                  carry=carry)(body)

    bufs = [(pbufA, tbufA, semPA, semTA), (pbufB, tbufB, semPB, semTB)]

    def issue(ci, slot):
        pb, tb, sp, st = bufs[slot]
        row = base + ci * CR
        pltpu.async_copy(pred.at[pl.ds(row, CR), :], pb, sp)
        pltpu.async_copy(tgt.at[pl.ds(row, CR), :], tb, st)

    def drain(slot):
        pb, tb, sp, st = bufs[slot]
        pltpu.make_async_copy(pred.at[pl.ds(0, CR), :], pb, sp).wait()
        pltpu.make_async_copy(tgt.at[pl.ds(0, CR), :], tb, st).wait()

    issue(0, 0)
    issue(1, 1)
    zi = jnp.zeros((L,), jnp.int32)
    carry = (z16, one16, zi, one16, zi)

    def pair_body(j, c):
        drain(0)
        c = process(pbufA, tbufA, c)
        issue(jnp.minimum(2 * j + 2, FNCHUNK - 1), 0)
        drain(1)
        c = process(pbufB, tbufB, c)
        issue(jnp.minimum(2 * j + 3, FNCHUNK - 1), 1)
        return c

    carry = lax.fori_loop(0, FNCHUNK // 2, pair_body, carry)
    drain(0)
    drain(1)
    posc, mn, en, mp, ep = carry

    negs = -_ln_mant(mn, (en - EBIAS).astype(jnp.float32))
    poss = -_ln_mant(mp, (ep - EBIAS).astype(jnp.float32))
    alls = poss + negs

    statv[pl.ds(0, L)] = posc
    statv[pl.ds(16, L)] = poss
    statv[pl.ds(32, L)] = alls
    for v in range(3, 8):
        statv[pl.ds(v * 16, L)] = z16
    pltpu.sync_copy(statv, sh_stats.at[pl.ds(sid * 128, 128)])
    plsc.subcore_barrier()

    @pl.when(sid == 0)
    def _():
        pltpu.sync_copy(sh_stats, stage.at[pl.ds(0, NS * 128)])
        for v in range(3):
            acc = z16
            for r in range(NS):
                acc = acc + stage[pl.ds(r * 128 + v * 16, L)]
            statv[pl.ds(v * 16, L)] = acc
        pltpu.sync_copy(statv, outst.at[pl.ds(cid * 128, 128)])


@functools.partial(
    pl.kernel,
    out_type=(
        jax.ShapeDtypeStruct((NC * NB,), jnp.float32),
        jax.ShapeDtypeStruct((NC * NB,), jnp.float32),
        jax.ShapeDtypeStruct((NC * 128,), jnp.float32),
    ),
    mesh=_mesh,
    compiler_params=_cparams_tc,
    scratch_types=[
        pltpu.VMEM((CR, COLS), jnp.float32),
        pltpu.VMEM((CR, COLS), jnp.float32),
        pltpu.VMEM((CR, COLS), jnp.float32),
        pltpu.VMEM((CR, COLS), jnp.float32),
        pltpu.VMEM((NB,), jnp.float32),
        pltpu.VMEM((NB,), jnp.float32),
        pltpu.VMEM((BB,), jnp.float32),
        pltpu.VMEM((BB,), jnp.float32),
        pltpu.VMEM((2048,), jnp.float32),
        pltpu.VMEM((128,), jnp.float32),
        pltpu.VMEM_SHARED((NS * NB,), jnp.float32),
        pltpu.VMEM_SHARED((NS * NB,), jnp.float32),
        pltpu.VMEM_SHARED((NB,), jnp.float32),
        pltpu.VMEM_SHARED((NB,), jnp.float32),
        pltpu.VMEM_SHARED((NS * 128,), jnp.float32),
        pltpu.SemaphoreType.DMA,
        pltpu.SemaphoreType.DMA,
        pltpu.SemaphoreType.DMA,
        pltpu.SemaphoreType.DMA,
    ],
)
def _k1(pred, tgt, outc, outs, outst,
        pbufA, tbufA, pbufB, tbufB, histc_v, hists_v, accc, accs, stage, statv,
        sh_histc, sh_hists, sh_redc, sh_reds, sh_stats,
        semPA, semTA, semPB, semTB):
    cid = lax.axis_index("c")
    sid = lax.axis_index("s")
    base = (cid * NS + sid) * ROWS_W
    z16 = jnp.zeros((L,), jnp.float32)
    ones16 = jnp.full((L,), 1.0, jnp.float32)

    def _zi(i, _):
        o = pl.multiple_of(i * L, L)
        histc_v[pl.ds(o, L)] = z16
        hists_v[pl.ds(o, L)] = z16
        return 0
    lax.fori_loop(0, NB // L, _zi, 0)

    def process(pref, tref, carry):
        def body(j, c):
            posc, poss, alls = c
            r = j >> 5
            o = pl.multiple_of((j & 31) * L, L)
            if True:
                p = pref[r, pl.ds(o, L)]
                t = tref[r, pl.ds(o, L)]
                loss, pos = _loss16(p, t)
                alls = alls + loss
                posc = posc + t
                poss = poss + jnp.where(pos, loss, jnp.float32(0.0))
                binv = jnp.minimum(
                    (loss * jnp.float32(SCALE)).astype(jnp.int32), NB - 1)
                negm = jnp.logical_not(pos)
                plsc.addupdate_scatter(histc_v, [binv], ones16, mask=negm)
                plsc.addupdate_scatter(hists_v, [binv], loss, mask=negm)
            return posc, poss, alls
        return plsc.parallel_loop(0, CR * COLS // L, unroll=4,
                                  carry=carry)(body)

    bufs = [(pbufA, tbufA, semPA, semTA), (pbufB, tbufB, semPB, semTB)]

    def issue(ci, slot):
        pb, tb, sp, st = bufs[slot]
        row = base + ci * CR
        pltpu.async_copy(pred.at[pl.ds(row, CR), :], pb, sp)
        pltpu.async_copy(tgt.at[pl.ds(row, CR), :], tb, st)

    def drain(slot):
        pb, tb, sp, st = bufs[slot]
        pltpu.make_async_copy(pred.at[pl.ds(0, CR), :], pb, sp).wait()
        pltpu.make_async_copy(tgt.at[pl.ds(0, CR), :], tb, st).wait()

    issue(0, 0)
    issue(1, 1)
    carry = (z16, z16, z16)

    def pair_body(j, c):
        drain(0)
        c = process(pbufA, tbufA, c)
        issue(jnp.minimum(2 * j + 2, NCHUNK - 1), 0)
        drain(1)
        c = process(pbufB, tbufB, c)
        issue(jnp.minimum(2 * j + 3, NCHUNK - 1), 1)
        return c

    carry = lax.fori_loop(0, NCHUNK // 2, pair_body, carry)
    drain(0)
    drain(1)
    posc, poss, alls = carry

    statv[pl.ds(0, L)] = posc
    statv[pl.ds(16, L)] = poss
    statv[pl.ds(32, L)] = alls
    for v in range(3, 8):
        statv[pl.ds(v * 16, L)] = z16
    pltpu.sync_copy(statv, sh_stats.at[pl.ds(sid * 128, 128)])
    pltpu.sync_copy(histc_v, sh_histc.at[pl.ds(sid * NB, NB)])
    pltpu.sync_copy(hists_v, sh_hists.at[pl.ds(sid * NB, NB)])
    plsc.subcore_barrier()

    for r in range(NS):
        pltpu.sync_copy(sh_histc.at[pl.ds(r * NB + sid * BB, BB)],
                        histc_v.at[pl.ds(r * BB, BB)])
        pltpu.sync_copy(sh_hists.at[pl.ds(r * NB + sid * BB, BB)],
                        hists_v.at[pl.ds(r * BB, BB)])

    def _red(v, _):
        o = pl.multiple_of(v * L, L)
        cacc = z16
        sacc = z16
        for r in range(NS):
            cacc = cacc + histc_v[pl.ds(r * BB + o, L)]
            sacc = sacc + hists_v[pl.ds(r * BB + o, L)]
        accc[pl.ds(o, L)] = cacc
        accs[pl.ds(o, L)] = sacc
        return 0
    lax.fori_loop(0, BB // L, _red, 0)
    pltpu.sync_copy(accc, sh_redc.at[pl.ds(sid * BB, BB)])
    pltpu.sync_copy(accs, sh_reds.at[pl.ds(sid * BB, BB)])
    plsc.subcore_barrier()

    @pl.when(sid == 0)
    def _():
        pltpu.sync_copy(sh_redc, outc.at[pl.ds(cid * NB, NB)])
        pltpu.sync_copy(sh_reds, outs.at[pl.ds(cid * NB, NB)])
        pltpu.sync_copy(sh_stats, stage.at[pl.ds(0, NS * 128)])
        for v in range(3):
            acc = z16
            for r in range(NS):
                acc = acc + stage[pl.ds(r * 128 + v * 16, L)]
            statv[pl.ds(v * 16, L)] = acc
        pltpu.sync_copy(statv, outst.at[pl.ds(cid * 128, 128)])


@functools.partial(
    pl.kernel,
    out_type=jax.ShapeDtypeStruct((L,), jnp.float32),
    mesh=_mesh,
    compiler_params=_cparams,
    scratch_types=[
        pltpu.VMEM((NB,), jnp.float32),
        pltpu.VMEM((NB,), jnp.float32),
        pltpu.VMEM((NB,), jnp.float32),
        pltpu.VMEM((NB,), jnp.float32),
        pltpu.VMEM((256,), jnp.float32),
        pltpu.VMEM((L,), jnp.float32),
    ],
)
def _k2(histc, hists, stats, out, c0, c1, s0, s1, st_v, outbuf):
    cid = lax.axis_index("c")
    sid = lax.axis_index("s")

    @pl.when(jnp.logical_and(cid == 0, sid == 0))
    def _():
        pltpu.sync_copy(histc.at[pl.ds(0, NB)], c0)
        pltpu.sync_copy(histc.at[pl.ds(NB, NB)], c1)
        pltpu.sync_copy(hists.at[pl.ds(0, NB)], s0)
        pltpu.sync_copy(hists.at[pl.ds(NB, NB)], s1)
        pltpu.sync_copy(stats, st_v)

        pos_cnt = jnp.sum(st_v[pl.ds(0, L)] + st_v[pl.ds(128, L)])
        pos_sum = jnp.sum(st_v[pl.ds(16, L)] + st_v[pl.ds(144, L)])
        all_sum = jnp.sum(st_v[pl.ds(32, L)] + st_v[pl.ds(160, L)])
        neg_cnt = jnp.float32(N) - pos_cnt
        neg_sum = all_sum - pos_sum
        k = jnp.minimum(neg_cnt, jnp.float32(3.0) * pos_cnt)

        def sel_body(jj, carry):
            above, sel = carry
            o = (NB // L - 1 - jj) * L
            cv = c0[pl.ds(o, L)] + c1[pl.ds(o, L)]
            sv = s0[pl.ds(o, L)] + s1[pl.ds(o, L)]
            pc = plsc.cumsum(cv)
            tot = jnp.sum(cv)
            above_i = above + (tot - pc)
            take = jnp.minimum(jnp.maximum(k - above_i, jnp.float32(0.0)), cv)
            avg = sv / jnp.maximum(cv, jnp.float32(1.0))
            sel = sel + jnp.sum(take * avg)
            return above + tot, sel

        _, sel = lax.fori_loop(0, NB // L, sel_body,
                               (jnp.float32(0.0), jnp.float32(0.0)))
        neg_loss = jnp.where(k >= neg_cnt, neg_sum, sel)
        total = pos_cnt + k
        ones_v = jnp.full((L,), 1.0, jnp.float32)
        num_v = ones_v * (pos_sum + neg_loss)
        den_v = ones_v * jnp.maximum(total, jnp.float32(1.0))
        res_v = num_v / den_v
        outbuf[...] = jnp.where(ones_v * total > jnp.float32(0.0), res_v,
                                jnp.zeros((L,), jnp.float32))
        pltpu.sync_copy(outbuf, out)


_TCG = 8
_TBR = TC_ROWS // _TCG


def _tc_body(x_ref, t_ref, oc_ref, op_ref, oa_ref):
    i = pl.program_id(0)

    @pl.when(i == 0)
    def _():
        oc_ref[...] = jnp.zeros_like(oc_ref)
        op_ref[...] = jnp.zeros_like(op_ref)
        oa_ref[...] = jnp.zeros_like(oa_ref)

    p = x_ref[...]
    t = t_ref[...]
    pc = jnp.clip(p, jnp.float32(EPS), jnp.float32(1.0) - jnp.float32(EPS))
    pos = t > jnp.float32(0.5)
    q = jnp.where(pos, pc, jnp.float32(1.0) - pc)
    loss = -jnp.log(q)
    oc_ref[...] += jnp.sum(t).reshape(1, 1)
    op_ref[...] += jnp.sum(jnp.where(pos, loss, jnp.float32(0.0))).reshape(1, 1)
    oa_ref[...] += jnp.sum(loss).reshape(1, 1)


_s11 = jax.ShapeDtypeStruct((1, 1), jnp.float32)
_tcsum = pl.pallas_call(
    _tc_body,
    grid=(_TCG,),
    in_specs=[
        pl.BlockSpec((_TBR, COLS), lambda i: (SC_ROWS // _TBR + i, 0)),
        pl.BlockSpec((_TBR, COLS), lambda i: (SC_ROWS // _TBR + i, 0)),
    ],
    out_specs=(
        pl.BlockSpec((1, 1), lambda i: (0, 0)),
        pl.BlockSpec((1, 1), lambda i: (0, 0)),
        pl.BlockSpec((1, 1), lambda i: (0, 0)),
    ),
    out_shape=(_s11, _s11, _s11),
)


def kernel(pred, target):
    p = pred.reshape(ROWS, COLS)
    t = target.reshape(ROWS, COLS)
    stats = _k1f(p, t).reshape(NC, 128)
    tc_cnt, tc_pos, tc_all = _tcsum(p, t)
    pos_cnt = jnp.sum(stats[:, 0:16]) + tc_cnt[0, 0]
    pos_sum = jnp.sum(stats[:, 16:32]) + tc_pos[0, 0]
    all_sum = jnp.sum(stats[:, 32:48]) + tc_all[0, 0]
    neg_cnt = jnp.float32(N) - pos_cnt
    neg_sum = all_sum - pos_sum
    k = jnp.minimum(neg_cnt, jnp.floor(jnp.float32(3.0) * pos_cnt))
    res = (pos_sum + neg_sum) / jnp.float32(N)
    need_sel = k < neg_cnt

    def _slow():
        hc, hs, st = _k1(p, t)
        return _k2(hc, hs, st)[0]

    return lax.cond(need_sel, _slow, lambda: res)


# speedup vs baseline: 1.2276x; 1.0094x over previous
import functools

import jax
import jax.numpy as jnp
from jax import lax
from jax.experimental import pallas as pl
from jax.experimental.pallas import tpu as pltpu
from jax.experimental.pallas import tpu_sc as plsc

N = 16 * 1 * 512 * 512
NC, NS, L = 2, 16, 16
NW = NC * NS
ROWS, COLS = 8192, 512
SC_ROWS = 2048
TC_ROWS = ROWS - SC_ROWS
ROWS_W = ROWS // NW
FROWS_W = SC_ROWS // NW
CR = 32
NCHUNK = ROWS_W // CR
FNCHUNK = FROWS_W // CR
GRP = 4
RENORMS = FROWS_W * (COLS // (GRP * L))
EBIAS = RENORMS * 127
NB = 1024
BB = NB // NS
LMAX = 16.2
SCALE = NB / LMAX
LN2 = 0.6931471805599453
EPS = 1e-7
SQRT2 = 1.4142135
MANT = 0x007FFFFF
ONEB = 0x3F800000
_LOGC = (6.43456457838365e-08, 1.0000040910390389, -0.5000199361111282,
         0.33299593064817884, -0.24886355774399765, 0.20655376876344744,
         -0.18852653680148798, 0.11589704819807638)

_mesh = plsc.VectorSubcoreMesh(
    core_axis_name="c", subcore_axis_name="s", num_cores=NC, num_subcores=NS)
_cparams = pltpu.CompilerParams(needs_layout_passes=False)
_cparams_tc = pltpu.CompilerParams(needs_layout_passes=False,
                                   use_tc_tiling_on_sc=True)


def _ln_mant(m, e_f):
    one = jnp.float32(1.0)
    big = m > jnp.float32(SQRT2)
    m = jnp.where(big, m * jnp.float32(0.5), m)
    e_f = e_f + jnp.where(big, one, jnp.float32(0.0))
    s = m - one
    pp = jnp.float32(_LOGC[7])
    for c in _LOGC[6::-1]:
        pp = pp * s + jnp.float32(c)
    return e_f * jnp.float32(LN2) + pp


def _loss16(p, t):
    one = jnp.float32(1.0)
    pc = jnp.minimum(jnp.maximum(p, jnp.float32(EPS)), one - jnp.float32(EPS))
    pos = t > jnp.float32(0.5)
    q = jnp.where(pos, pc, one - pc)
    bits = lax.bitcast_convert_type(q, jnp.int32)
    e = (bits >> 23) - 127
    m = lax.bitcast_convert_type((bits & MANT) | ONEB, jnp.float32)
    lnq = _ln_mant(m, e.astype(jnp.float32))
    return -lnq, pos


@functools.partial(
    pl.kernel,
    out_type=jax.ShapeDtypeStruct((NC * 128,), jnp.float32),
    mesh=_mesh,
    compiler_params=_cparams_tc,
    scratch_types=[
        pltpu.VMEM((CR, COLS), jnp.float32),
        pltpu.VMEM((CR, COLS), jnp.float32),
        pltpu.VMEM((CR, COLS), jnp.float32),
        pltpu.VMEM((CR, COLS), jnp.float32),
        pltpu.VMEM((2048,), jnp.float32),
        pltpu.VMEM((128,), jnp.float32),
        pltpu.VMEM_SHARED((NS * 128,), jnp.float32),
        pltpu.SemaphoreType.DMA,
        pltpu.SemaphoreType.DMA,
        pltpu.SemaphoreType.DMA,
        pltpu.SemaphoreType.DMA,
    ],
)
def _k1f(pred, tgt, outst,
         pbufA, tbufA, pbufB, tbufB, stage, statv, sh_stats,
         semPA, semTA, semPB, semTB):
    cid = lax.axis_index("c")
    sid = lax.axis_index("s")
    base = (cid * NS + sid) * FROWS_W
    z16 = jnp.zeros((L,), jnp.float32)
    one16 = jnp.full((L,), 1.0, jnp.float32)
    one = jnp.float32(1.0)

    def process(pref, tref, carry):
        def body(i, c):
            posc, mn, en, mp, ep = c
            r = i >> 3
            o = pl.multiple_of((i & 7) * (GRP * L), GRP * L)
            if True:
                qns, qps, ts = [], [], []
                for v in range(GRP):
                    p = pref[r, pl.ds(o + v * L, L)]
                    t = tref[r, pl.ds(o + v * L, L)]
                    pos = t > jnp.float32(0.5)
                    qps.append(jnp.maximum(jnp.where(pos, p, one16),
                                           jnp.float32(EPS)))
                    qns.append(jnp.maximum(jnp.where(pos, one16, one - p),
                                           jnp.float32(EPS)))
                    ts.append(t)
                posc = posc + ((ts[0] + ts[1]) + (ts[2] + ts[3]))
                pqn = (qns[0] * qns[1]) * (qns[2] * qns[3])
                pqp = (qps[0] * qps[1]) * (qps[2] * qps[3])
                mn = mn * pqn
                mp = mp * pqp
                bn = lax.bitcast_convert_type(mn, jnp.int32)
                en = en + (bn >> 23)
                mn = lax.bitcast_convert_type((bn & MANT) | ONEB, jnp.float32)
                bp = lax.bitcast_convert_type(mp, jnp.int32)
                ep = ep + (bp >> 23)
                mp = lax.bitcast_convert_type((bp & MANT) | ONEB, jnp.float32)
            return posc, mn, en, mp, ep
        return plsc.parallel_loop(0, CR * COLS // (GRP * L), unroll=2,
                                  carry=carry)(body)

    bufs = [(pbufA, tbufA, semPA, semTA), (pbufB, tbufB, semPB, semTB)]

    def issue(ci, slot):
        pb, tb, sp, st = bufs[slot]
        row = base + ci * CR
        pltpu.async_copy(pred.at[pl.ds(row, CR), :], pb, sp)
        pltpu.async_copy(tgt.at[pl.ds(row, CR), :], tb, st)

    def drain(slot):
        pb, tb, sp, st = bufs[slot]
        pltpu.make_async_copy(pred.at[pl.ds(0, CR), :], pb, sp).wait()
        pltpu.make_async_copy(tgt.at[pl.ds(0, CR), :], tb, st).wait()

    issue(0, 0)
    issue(1, 1)
    zi = jnp.zeros((L,), jnp.int32)
    carry = (z16, one16, zi, one16, zi)

    def pair_body(j, c):
        drain(0)
        c = process(pbufA, tbufA, c)
        issue(jnp.minimum(2 * j + 2, FNCHUNK - 1), 0)
        drain(1)
        c = process(pbufB, tbufB, c)
        issue(jnp.minimum(2 * j + 3, FNCHUNK - 1), 1)
        return c

    carry = lax.fori_loop(0, FNCHUNK // 2, pair_body, carry)
    drain(0)
    drain(1)
    posc, mn, en, mp, ep = carry

    negs = -_ln_mant(mn, (en - EBIAS).astype(jnp.float32))
    poss = -_ln_mant(mp, (ep - EBIAS).astype(jnp.float32))
    alls = poss + negs

    statv[pl.ds(0, L)] = posc
    statv[pl.ds(16, L)] = poss
    statv[pl.ds(32, L)] = alls
    for v in range(3, 8):
        statv[pl.ds(v * 16, L)] = z16
    pltpu.sync_copy(statv, sh_stats.at[pl.ds(sid * 128, 128)])
    plsc.subcore_barrier()

    @pl.when(sid == 0)
    def _():
        pltpu.sync_copy(sh_stats, stage.at[pl.ds(0, NS * 128)])
        for v in range(3):
            acc = z16
            for r in range(NS):
                acc = acc + stage[pl.ds(r * 128 + v * 16, L)]
            statv[pl.ds(v * 16, L)] = acc
        pltpu.sync_copy(statv, outst.at[pl.ds(cid * 128, 128)])


@functools.partial(
    pl.kernel,
    out_type=(
        jax.ShapeDtypeStruct((NC * NB,), jnp.float32),
        jax.ShapeDtypeStruct((NC * NB,), jnp.float32),
        jax.ShapeDtypeStruct((NC * 128,), jnp.float32),
    ),
    mesh=_mesh,
    compiler_params=_cparams_tc,
    scratch_types=[
        pltpu.VMEM((CR, COLS), jnp.float32),
        pltpu.VMEM((CR, COLS), jnp.float32),
        pltpu.VMEM((CR, COLS), jnp.float32),
        pltpu.VMEM((CR, COLS), jnp.float32),
        pltpu.VMEM((NB,), jnp.float32),
        pltpu.VMEM((NB,), jnp.float32),
        pltpu.VMEM((BB,), jnp.float32),
        pltpu.VMEM((BB,), jnp.float32),
        pltpu.VMEM((2048,), jnp.float32),
        pltpu.VMEM((128,), jnp.float32),
        pltpu.VMEM_SHARED((NS * NB,), jnp.float32),
        pltpu.VMEM_SHARED((NS * NB,), jnp.float32),
        pltpu.VMEM_SHARED((NB,), jnp.float32),
        pltpu.VMEM_SHARED((NB,), jnp.float32),
        pltpu.VMEM_SHARED((NS * 128,), jnp.float32),
        pltpu.SemaphoreType.DMA,
        pltpu.SemaphoreType.DMA,
        pltpu.SemaphoreType.DMA,
        pltpu.SemaphoreType.DMA,
    ],
)
def _k1(pred, tgt, outc, outs, outst,
        pbufA, tbufA, pbufB, tbufB, histc_v, hists_v, accc, accs, stage, statv,
        sh_histc, sh_hists, sh_redc, sh_reds, sh_stats,
        semPA, semTA, semPB, semTB):
    cid = lax.axis_index("c")
    sid = lax.axis_index("s")
    base = (cid * NS + sid) * ROWS_W
    z16 = jnp.zeros((L,), jnp.float32)
    ones16 = jnp.full((L,), 1.0, jnp.float32)

    def _zi(i, _):
        o = pl.multiple_of(i * L, L)
        histc_v[pl.ds(o, L)] = z16
        hists_v[pl.ds(o, L)] = z16
        return 0
    lax.fori_loop(0, NB // L, _zi, 0)

    def process(pref, tref, carry):
        def body(j, c):
            posc, poss, alls = c
            r = j >> 5
            o = pl.multiple_of((j & 31) * L, L)
            if True:
                p = pref[r, pl.ds(o, L)]
                t = tref[r, pl.ds(o, L)]
                loss, pos = _loss16(p, t)
                alls = alls + loss
                posc = posc + t
                poss = poss + jnp.where(pos, loss, jnp.float32(0.0))
                binv = jnp.minimum(
                    (loss * jnp.float32(SCALE)).astype(jnp.int32), NB - 1)
                negm = jnp.logical_not(pos)
                plsc.addupdate_scatter(histc_v, [binv], ones16, mask=negm)
                plsc.addupdate_scatter(hists_v, [binv], loss, mask=negm)
            return posc, poss, alls
        return plsc.parallel_loop(0, CR * COLS // L, unroll=4,
                                  carry=carry)(body)

    bufs = [(pbufA, tbufA, semPA, semTA), (pbufB, tbufB, semPB, semTB)]

    def issue(ci, slot):
        pb, tb, sp, st = bufs[slot]
        row = base + ci * CR
        pltpu.async_copy(pred.at[pl.ds(row, CR), :], pb, sp)
        pltpu.async_copy(tgt.at[pl.ds(row, CR), :], tb, st)

    def drain(slot):
        pb, tb, sp, st = bufs[slot]
        pltpu.make_async_copy(pred.at[pl.ds(0, CR), :], pb, sp).wait()
        pltpu.make_async_copy(tgt.at[pl.ds(0, CR), :], tb, st).wait()

    issue(0, 0)
    issue(1, 1)
    carry = (z16, z16, z16)

    def pair_body(j, c):
        drain(0)
        c = process(pbufA, tbufA, c)
        issue(jnp.minimum(2 * j + 2, NCHUNK - 1), 0)
        drain(1)
        c = process(pbufB, tbufB, c)
        issue(jnp.minimum(2 * j + 3, NCHUNK - 1), 1)
        return c

    carry = lax.fori_loop(0, NCHUNK // 2, pair_body, carry)
    drain(0)
    drain(1)
    posc, poss, alls = carry

    statv[pl.ds(0, L)] = posc
    statv[pl.ds(16, L)] = poss
    statv[pl.ds(32, L)] = alls
    for v in range(3, 8):
        statv[pl.ds(v * 16, L)] = z16
    pltpu.sync_copy(statv, sh_stats.at[pl.ds(sid * 128, 128)])
    pltpu.sync_copy(histc_v, sh_histc.at[pl.ds(sid * NB, NB)])
    pltpu.sync_copy(hists_v, sh_hists.at[pl.ds(sid * NB, NB)])
    plsc.subcore_barrier()

    for r in range(NS):
        pltpu.sync_copy(sh_histc.at[pl.ds(r * NB + sid * BB, BB)],
                        histc_v.at[pl.ds(r * BB, BB)])
        pltpu.sync_copy(sh_hists.at[pl.ds(r * NB + sid * BB, BB)],
                        hists_v.at[pl.ds(r * BB, BB)])

    def _red(v, _):
        o = pl.multiple_of(v * L, L)
        cacc = z16
        sacc = z16
        for r in range(NS):
            cacc = cacc + histc_v[pl.ds(r * BB + o, L)]
            sacc = sacc + hists_v[pl.ds(r * BB + o, L)]
        accc[pl.ds(o, L)] = cacc
        accs[pl.ds(o, L)] = sacc
        return 0
    lax.fori_loop(0, BB // L, _red, 0)
    pltpu.sync_copy(accc, sh_redc.at[pl.ds(sid * BB, BB)])
    pltpu.sync_copy(accs, sh_reds.at[pl.ds(sid * BB, BB)])
    plsc.subcore_barrier()

    @pl.when(sid == 0)
    def _():
        pltpu.sync_copy(sh_redc, outc.at[pl.ds(cid * NB, NB)])
        pltpu.sync_copy(sh_reds, outs.at[pl.ds(cid * NB, NB)])
        pltpu.sync_copy(sh_stats, stage.at[pl.ds(0, NS * 128)])
        for v in range(3):
            acc = z16
            for r in range(NS):
                acc = acc + stage[pl.ds(r * 128 + v * 16, L)]
            statv[pl.ds(v * 16, L)] = acc
        pltpu.sync_copy(statv, outst.at[pl.ds(cid * 128, 128)])


@functools.partial(
    pl.kernel,
    out_type=jax.ShapeDtypeStruct((L,), jnp.float32),
    mesh=_mesh,
    compiler_params=_cparams,
    scratch_types=[
        pltpu.VMEM((NB,), jnp.float32),
        pltpu.VMEM((NB,), jnp.float32),
        pltpu.VMEM((NB,), jnp.float32),
        pltpu.VMEM((NB,), jnp.float32),
        pltpu.VMEM((256,), jnp.float32),
        pltpu.VMEM((L,), jnp.float32),
    ],
)
def _k2(histc, hists, stats, out, c0, c1, s0, s1, st_v, outbuf):
    cid = lax.axis_index("c")
    sid = lax.axis_index("s")

    @pl.when(jnp.logical_and(cid == 0, sid == 0))
    def _():
        pltpu.sync_copy(histc.at[pl.ds(0, NB)], c0)
        pltpu.sync_copy(histc.at[pl.ds(NB, NB)], c1)
        pltpu.sync_copy(hists.at[pl.ds(0, NB)], s0)
        pltpu.sync_copy(hists.at[pl.ds(NB, NB)], s1)
        pltpu.sync_copy(stats, st_v)

        pos_cnt = jnp.sum(st_v[pl.ds(0, L)] + st_v[pl.ds(128, L)])
        pos_sum = jnp.sum(st_v[pl.ds(16, L)] + st_v[pl.ds(144, L)])
        all_sum = jnp.sum(st_v[pl.ds(32, L)] + st_v[pl.ds(160, L)])
        neg_cnt = jnp.float32(N) - pos_cnt
        neg_sum = all_sum - pos_sum
        k = jnp.minimum(neg_cnt, jnp.float32(3.0) * pos_cnt)

        def sel_body(jj, carry):
            above, sel = carry
            o = (NB // L - 1 - jj) * L
            cv = c0[pl.ds(o, L)] + c1[pl.ds(o, L)]
            sv = s0[pl.ds(o, L)] + s1[pl.ds(o, L)]
            pc = plsc.cumsum(cv)
            tot = jnp.sum(cv)
            above_i = above + (tot - pc)
            take = jnp.minimum(jnp.maximum(k - above_i, jnp.float32(0.0)), cv)
            avg = sv / jnp.maximum(cv, jnp.float32(1.0))
            sel = sel + jnp.sum(take * avg)
            return above + tot, sel

        _, sel = lax.fori_loop(0, NB // L, sel_body,
                               (jnp.float32(0.0), jnp.float32(0.0)))
        neg_loss = jnp.where(k >= neg_cnt, neg_sum, sel)
        total = pos_cnt + k
        ones_v = jnp.full((L,), 1.0, jnp.float32)
        num_v = ones_v * (pos_sum + neg_loss)
        den_v = ones_v * jnp.maximum(total, jnp.float32(1.0))
        res_v = num_v / den_v
        outbuf[...] = jnp.where(ones_v * total > jnp.float32(0.0), res_v,
                                jnp.zeros((L,), jnp.float32))
        pltpu.sync_copy(outbuf, out)


_TCG = 8
_TBR = TC_ROWS // _TCG


def _tc_body(x_ref, t_ref, oc_ref, op_ref, oa_ref):
    i = pl.program_id(0)

    @pl.when(i == 0)
    def _():
        oc_ref[...] = jnp.zeros_like(oc_ref)
        op_ref[...] = jnp.zeros_like(op_ref)
        oa_ref[...] = jnp.zeros_like(oa_ref)

    p = x_ref[...]
    t = t_ref[...]
    pc = jnp.clip(p, jnp.float32(EPS), jnp.float32(1.0) - jnp.float32(EPS))
    pos = t > jnp.float32(0.5)
    q = jnp.where(pos, pc, jnp.float32(1.0) - pc)
    loss = -jnp.log(q)
    oc_ref[...] += jnp.sum(jnp.sum(t, axis=0)).reshape(1, 1)
    op_ref[...] += jnp.sum(jnp.sum(jnp.where(pos, loss, jnp.float32(0.0)),
                                   axis=0)).reshape(1, 1)
    oa_ref[...] += jnp.sum(jnp.sum(loss, axis=0)).reshape(1, 1)


_s11 = jax.ShapeDtypeStruct((1, 1), jnp.float32)
_tcsum = pl.pallas_call(
    _tc_body,
    grid=(_TCG,),
    in_specs=[
        pl.BlockSpec((_TBR, COLS), lambda i: (SC_ROWS // _TBR + i, 0)),
        pl.BlockSpec((_TBR, COLS), lambda i: (SC_ROWS // _TBR + i, 0)),
    ],
    out_specs=(
        pl.BlockSpec((1, 1), lambda i: (0, 0)),
        pl.BlockSpec((1, 1), lambda i: (0, 0)),
        pl.BlockSpec((1, 1), lambda i: (0, 0)),
    ),
    out_shape=(_s11, _s11, _s11),
)


def kernel(pred, target):
    p = pred.reshape(ROWS, COLS)
    t = target.reshape(ROWS, COLS)
    stats = _k1f(p, t).reshape(NC, 128)
    tc_cnt, tc_pos, tc_all = _tcsum(p, t)
    pos_cnt = jnp.sum(stats[:, 0:16]) + tc_cnt[0, 0]
    pos_sum = jnp.sum(stats[:, 16:32]) + tc_pos[0, 0]
    all_sum = jnp.sum(stats[:, 32:48]) + tc_all[0, 0]
    neg_cnt = jnp.float32(N) - pos_cnt
    neg_sum = all_sum - pos_sum
    k = jnp.minimum(neg_cnt, jnp.floor(jnp.float32(3.0) * pos_cnt))
    res = (pos_sum + neg_sum) / jnp.float32(N)
    need_sel = k < neg_cnt

    def _slow():
        hc, hs, st = _k1(p, t)
        return _k2(hc, hs, st)[0]

    return lax.cond(need_sel, _slow, lambda: res)
